# baseline (speedup 1.0000x reference)
import functools

import jax
import jax.numpy as jnp
from jax import lax
from jax.experimental import pallas as pl
from jax.experimental.pallas import tpu as pltpu
from jax.experimental.pallas import tpu_sc as plsc

HID = 64
NB = 4096
SEQ = 200
N = NB * SEQ
NC = 2
NS = 16
NW = NC * NS
RPW = N // NW
CHUNK = 256
NCHUNK = RPW // CHUNK
ISTREAM = 128
NSTREAM = CHUNK // ISTREAM
EPS = 1e-12


def _rsqrt(x):
    i = lax.bitcast_convert_type(x, jnp.int32)
    i = jnp.int32(0x5F3759DF) - (i >> 1)
    y = lax.bitcast_convert_type(i, jnp.float32)
    for _ in range(3):
        y = y * (1.5 - 0.5 * x * y * y)
    return y


def _fire_gather(tab, idx_all, rows, sem, c):
    for j in range(NSTREAM):
        pltpu.async_copy(
            tab.at[idx_all.at[pl.ds(c * CHUNK + j * ISTREAM, ISTREAM)]],
            rows.at[pl.ds(j * ISTREAM, ISTREAM)],
            sem,
        )


def _wait_gather(tab, idx_all, rows, sem, c):
    for j in range(NSTREAM):
        pltpu.make_async_copy(
            tab.at[idx_all.at[pl.ds(c * CHUNK + j * ISTREAM, ISTREAM)]],
            rows.at[pl.ds(j * ISTREAM, ISTREAM)],
            sem,
        ).wait()


def _fire_scatter(rows, out, sem, wbase, c):
    pltpu.async_copy(rows, out.at[pl.ds(wbase + c * CHUNK, CHUNK)], sem)


def _wait_scatter(rows, out, sem, wbase, c):
    pltpu.make_async_copy(
        rows, out.at[pl.ds(wbase + c * CHUNK, CHUNK)], sem
    ).wait()


def _compute_chunk(rows, pos_v, w4, b4, c):
    @plsc.parallel_loop(0, CHUNK, 1, unroll=4)
    def _row(r):
        pr = lax.rem(c * CHUNK + r, jnp.int32(SEQ))
        h = []
        for k in range(4):
            a = rows[r, pl.ds(k * 16, 16)]
            p = pos_v[pr, pl.ds(k * 16, 16)]
            h.append(a + p)
        tot = (h[0] + h[1]) + (h[2] + h[3])
        su = jnp.sum(tot)
        sq = (h[0] * h[0] + h[1] * h[1]) + (h[2] * h[2] + h[3] * h[3])
        ssq = jnp.sum(sq)
        u = su * (1.0 / HID)
        var = ssq * (1.0 / HID) - u * u
        rinv = _rsqrt(jnp.maximum(var, 0.0) + EPS)
        for k in range(4):
            z = (h[k] - u) * rinv
            rows[r, pl.ds(k * 16, 16)] = z * w4[k] + b4[k]


def _body(
    x_hbm,
    tab,
    pos_hbm,
    w_hbm,
    b_hbm,
    out,
    idx_all,
    pos_v,
    rows_a,
    rows_b,
    wv,
    bv,
    gsa,
    gsb,
    osa,
    osb,
):
    wid = lax.axis_index("s") * NC + lax.axis_index("c")
    wbase = wid * RPW
    pltpu.sync_copy(x_hbm.at[pl.ds(wbase, RPW)], idx_all)
    pltpu.sync_copy(pos_hbm, pos_v)
    pltpu.sync_copy(w_hbm, wv)
    pltpu.sync_copy(b_hbm, bv)
    w4 = [wv[pl.ds(k * 16, 16)] for k in range(4)]
    b4 = [bv[pl.ds(k * 16, 16)] for k in range(4)]
    _fire_gather(tab, idx_all, rows_a, gsa, 0)

    @pl.loop(0, NCHUNK // 2)
    def _pair(cc):
        c0 = cc * 2
        c1 = c0 + 1

        @pl.when(cc > 0)
        def _():
            _wait_scatter(rows_b, out, osb, wbase, c1 - 2)

        _fire_gather(tab, idx_all, rows_b, gsb, c1)
        _wait_gather(tab, idx_all, rows_a, gsa, c0)
        _compute_chunk(rows_a, pos_v, w4, b4, c0)
        _fire_scatter(rows_a, out, osa, wbase, c0)

        @pl.when(cc < NCHUNK // 2 - 1)
        def _():
            _wait_scatter(rows_a, out, osa, wbase, c0)
            _fire_gather(tab, idx_all, rows_a, gsa, c0 + 2)

        _wait_gather(tab, idx_all, rows_b, gsb, c1)
        _compute_chunk(rows_b, pos_v, w4, b4, c1)
        _fire_scatter(rows_b, out, osb, wbase, c1)

    _wait_scatter(rows_a, out, osa, wbase, NCHUNK - 2)
    _wait_scatter(rows_b, out, osb, wbase, NCHUNK - 1)


@jax.jit
def kernel(x, alphabet_table, position_table, ln_weight, ln_bias):
    nb, seq = x.shape
    hid = alphabet_table.shape[1]
    assert (nb, seq, hid) == (NB, SEQ, HID)
    x_flat = x.reshape(N).astype(jnp.int32)
    pos = position_table[:SEQ]
    run = pl.kernel(
        _body,
        out_type=jax.ShapeDtypeStruct((N, HID), jnp.float32),
        mesh=plsc.VectorSubcoreMesh(core_axis_name="c", subcore_axis_name="s"),
        compiler_params=pltpu.CompilerParams(
            use_tc_tiling_on_sc=False, needs_layout_passes=False
        ),
        scratch_types=[
            pltpu.VMEM((RPW,), jnp.int32),
            pltpu.VMEM((SEQ, HID), jnp.float32),
            pltpu.VMEM((CHUNK, HID), jnp.float32),
            pltpu.VMEM((CHUNK, HID), jnp.float32),
            pltpu.VMEM((HID,), jnp.float32),
            pltpu.VMEM((HID,), jnp.float32),
            pltpu.SemaphoreType.DMA,
            pltpu.SemaphoreType.DMA,
            pltpu.SemaphoreType.DMA,
            pltpu.SemaphoreType.DMA,
        ],
    )
    out = run(x_flat, alphabet_table, pos, ln_weight, ln_bias)
    return out.reshape(NB, SEQ, HID)


# speedup vs baseline: 1.0294x; 1.0294x over previous
import functools

import jax
import jax.numpy as jnp
from jax import lax
from jax.experimental import pallas as pl
from jax.experimental.pallas import tpu as pltpu
from jax.experimental.pallas import tpu_sc as plsc

HID = 64
NB = 4096
SEQ = 200
NC = 2
NS = 16
NW = NC * NS
BPW = NB // NW
S0 = 104
EPS = 1e-12


def _rsqrt(x):
    i = lax.bitcast_convert_type(x, jnp.int32)
    i = jnp.int32(0x5F3759DF) - (i >> 1)
    y = lax.bitcast_convert_type(i, jnp.float32)
    for _ in range(3):
        y = y * (1.5 - 0.5 * x * y * y)
    return y


def _fire_gather(tab, idx2, rows, sem, c):
    for lo, ln in ((0, S0), (S0, SEQ - S0)):
        pltpu.async_copy(
            tab.at[idx2.at[c, pl.ds(lo, ln)]],
            rows.at[pl.ds(lo, ln)],
            sem,
        )


def _wait_gather(tab, idx2, rows, sem, c):
    for lo, ln in ((0, S0), (S0, SEQ - S0)):
        pltpu.make_async_copy(
            tab.at[idx2.at[c, pl.ds(lo, ln)]],
            rows.at[pl.ds(lo, ln)],
            sem,
        ).wait()


def _fire_scatter(rows, out, sem, b):
    pltpu.async_copy(rows, out.at[b], sem)


def _wait_scatter(rows, out, sem, b):
    pltpu.make_async_copy(rows, out.at[b], sem).wait()


def _compute_chunk(rows, pos_v, w4, b4):
    @plsc.parallel_loop(0, SEQ, 1, unroll=4)
    def _row(r):
        h = []
        for k in range(4):
            a = rows[r, pl.ds(k * 16, 16)]
            p = pos_v[r, pl.ds(k * 16, 16)]
            h.append(a + p)
        tot = (h[0] + h[1]) + (h[2] + h[3])
        su = jnp.sum(tot)
        sq = (h[0] * h[0] + h[1] * h[1]) + (h[2] * h[2] + h[3] * h[3])
        ssq = jnp.sum(sq)
        u = su * (1.0 / HID)
        var = ssq * (1.0 / HID) - u * u
        rinv = _rsqrt(jnp.maximum(var, 0.0) + EPS)
        for k in range(4):
            z = (h[k] - u) * rinv
            rows[r, pl.ds(k * 16, 16)] = z * w4[k] + b4[k]


def _body(
    x_hbm,
    tab,
    pos_hbm,
    w_hbm,
    b_hbm,
    out,
    idx2,
    pos_v,
    rows_a,
    rows_b,
    wv,
    bv,
    gsa,
    gsb,
    osa,
    osb,
):
    wid = lax.axis_index("s") * NC + lax.axis_index("c")
    b0 = wid * BPW
    pltpu.sync_copy(x_hbm.at[pl.ds(b0, BPW)], idx2)
    pltpu.sync_copy(pos_hbm.at[pl.ds(0, SEQ)], pos_v)
    pltpu.sync_copy(w_hbm, wv)
    pltpu.sync_copy(b_hbm, bv)
    w4 = [wv[pl.ds(k * 16, 16)] for k in range(4)]
    b4 = [bv[pl.ds(k * 16, 16)] for k in range(4)]
    _fire_gather(tab, idx2, rows_a, gsa, 0)

    @pl.loop(0, BPW // 2)
    def _pair(cc):
        c0 = cc * 2
        c1 = c0 + 1

        @pl.when(cc > 0)
        def _():
            _wait_scatter(rows_b, out, osb, b0 + c1 - 2)

        _fire_gather(tab, idx2, rows_b, gsb, c1)
        _wait_gather(tab, idx2, rows_a, gsa, c0)
        _compute_chunk(rows_a, pos_v, w4, b4)
        _fire_scatter(rows_a, out, osa, b0 + c0)

        @pl.when(cc < BPW // 2 - 1)
        def _():
            _wait_scatter(rows_a, out, osa, b0 + c0)
            _fire_gather(tab, idx2, rows_a, gsa, c0 + 2)

        _wait_gather(tab, idx2, rows_b, gsb, c1)
        _compute_chunk(rows_b, pos_v, w4, b4)
        _fire_scatter(rows_b, out, osb, b0 + c1)

    _wait_scatter(rows_a, out, osa, b0 + BPW - 2)
    _wait_scatter(rows_b, out, osb, b0 + BPW - 1)


@jax.jit
def kernel(x, alphabet_table, position_table, ln_weight, ln_bias):
    nb, seq = x.shape
    hid = alphabet_table.shape[1]
    assert (nb, seq, hid) == (NB, SEQ, HID)
    run = pl.kernel(
        _body,
        out_type=jax.ShapeDtypeStruct((NB, SEQ, HID), jnp.float32),
        mesh=plsc.VectorSubcoreMesh(core_axis_name="c", subcore_axis_name="s"),
        compiler_params=pltpu.CompilerParams(
            use_tc_tiling_on_sc=False, needs_layout_passes=False
        ),
        scratch_types=[
            pltpu.VMEM((BPW, SEQ), jnp.int32),
            pltpu.VMEM((SEQ, HID), jnp.float32),
            pltpu.VMEM((SEQ, HID), jnp.float32),
            pltpu.VMEM((SEQ, HID), jnp.float32),
            pltpu.VMEM((HID,), jnp.float32),
            pltpu.VMEM((HID,), jnp.float32),
            pltpu.SemaphoreType.DMA,
            pltpu.SemaphoreType.DMA,
            pltpu.SemaphoreType.DMA,
            pltpu.SemaphoreType.DMA,
        ],
    )
    return run(x, alphabet_table, position_table, ln_weight, ln_bias)
